# baseline (speedup 1.0000x reference)
import jax
import jax.numpy as jnp
from jax.experimental import pallas as pl
from jax.experimental.pallas import tpu as pltpu

_BS = 1024
_TW = 128


def kernel(x, y):
    del y
    B, S, D = x.shape
    R = B * S
    x2 = x.reshape(R, D)
    reps = D // _TW
    nsteps = R // _BS

    def copies_for(step, tile_ref, o_ref, sem):
        return [
            pltpu.make_async_copy(
                tile_ref,
                o_ref.at[pl.ds(step * _BS, _BS), pl.ds(k * _TW, _TW)],
                sem,
            )
            for k in range(reps)
        ]

    def body(x_ref, o_ref, t0, t1, s0, s1):
        i = pl.program_id(0)
        tiles, sems = (t0, t1), (s0, s1)

        def slot_ops(slot):
            tile, sem = tiles[slot], sems[slot]

            @pl.when(i >= 2)
            def _():
                for c in copies_for(i - 2, tile, o_ref, sem):
                    c.wait()

            tile[...] = jnp.broadcast_to(x_ref[:, 1:2], (_BS, _TW))
            for c in copies_for(i, tile, o_ref, sem):
                c.start()

        @pl.when(i % 2 == 0)
        def _():
            slot_ops(0)

        @pl.when(i % 2 == 1)
        def _():
            slot_ops(1)

        @pl.when(i == nsteps - 1)
        def _():
            for step in (nsteps - 2, nsteps - 1):
                slot = step % 2
                for c in copies_for(step, tiles[slot], o_ref, sems[slot]):
                    c.wait()

    out2 = pl.pallas_call(
        body,
        grid=(nsteps,),
        in_specs=[pl.BlockSpec((_BS, 128), lambda i: (i, 0))],
        out_specs=pl.BlockSpec(memory_space=pl.ANY),
        out_shape=jax.ShapeDtypeStruct((R, D), x.dtype),
        scratch_shapes=[
            pltpu.VMEM((_BS, _TW), x.dtype),
            pltpu.VMEM((_BS, _TW), x.dtype),
            pltpu.SemaphoreType.DMA,
            pltpu.SemaphoreType.DMA,
        ],
    )(x2)
    return out2.reshape(B, S, D)


# speedup vs baseline: 1.0016x; 1.0016x over previous
import jax
import jax.numpy as jnp
from jax.experimental import pallas as pl
from jax.experimental.pallas import tpu as pltpu

_BS = 1024
_TW = 256


def kernel(x, y):
    del y
    B, S, D = x.shape
    R = B * S
    x2 = x.reshape(R, D)
    reps = D // _TW
    nsteps = R // _BS

    def copies_for(step, tile_ref, o_ref, sem):
        return [
            pltpu.make_async_copy(
                tile_ref,
                o_ref.at[pl.ds(step * _BS, _BS), pl.ds(k * _TW, _TW)],
                sem,
            )
            for k in range(reps)
        ]

    def body(x_ref, o_ref, t0, t1, s0, s1):
        i = pl.program_id(0)
        tiles, sems = (t0, t1), (s0, s1)

        def slot_ops(slot):
            tile, sem = tiles[slot], sems[slot]

            @pl.when(i >= 2)
            def _():
                for c in copies_for(i - 2, tile, o_ref, sem):
                    c.wait()

            tile[...] = jnp.broadcast_to(x_ref[:, 1:2], (_BS, _TW))
            for c in copies_for(i, tile, o_ref, sem):
                c.start()

        @pl.when(i % 2 == 0)
        def _():
            slot_ops(0)

        @pl.when(i % 2 == 1)
        def _():
            slot_ops(1)

        @pl.when(i == nsteps - 1)
        def _():
            for step in (nsteps - 2, nsteps - 1):
                slot = step % 2
                for c in copies_for(step, tiles[slot], o_ref, sems[slot]):
                    c.wait()

    out2 = pl.pallas_call(
        body,
        grid=(nsteps,),
        in_specs=[pl.BlockSpec((_BS, 128), lambda i: (i, 0))],
        out_specs=pl.BlockSpec(memory_space=pl.ANY),
        out_shape=jax.ShapeDtypeStruct((R, D), x.dtype),
        scratch_shapes=[
            pltpu.VMEM((_BS, _TW), x.dtype),
            pltpu.VMEM((_BS, _TW), x.dtype),
            pltpu.SemaphoreType.DMA,
            pltpu.SemaphoreType.DMA,
        ],
    )(x2)
    return out2.reshape(B, S, D)
